# baseline (speedup 1.0000x reference)
import math

import jax
import jax.numpy as jnp
from jax import lax
from jax.experimental import pallas as pl
from jax.experimental.pallas import tpu as pltpu
from jax.experimental.pallas import tpu_sc as plsc

_NTOKENS = 1000000
_NSAMPLED = 8192
_NHID = 128
_BATCH = 4096
_NC, _NS = 2, 16
_NW = _NC * _NS
_L_PER = _BATCH // _NW
_S_PER = _NSAMPLED // _NW
_LOG_NT1 = math.log(_NTOKENS + 1)
_B_SLICE = _NTOKENS // 4


def _sc_gather_fn():
    f32, i32 = jnp.float32, jnp.int32
    mesh = plsc.VectorSubcoreMesh(
        core_axis_name="c", subcore_axis_name="s",
        num_cores=_NC, num_subcores=_NS)

    def body(w_hbm, b_hbm, lab_hbm, sid_hbm,
             tw_hbm, tb_hbm, sw_hbm, sb_hbm,
             idx_l, idx_s, rows_l, rows_s, bias_l, bias_s, sem, sem2):
        c = lax.axis_index("c")
        s = lax.axis_index("s")
        w = s * _NC + c
        pltpu.sync_copy(lab_hbm.at[pl.ds(w * _L_PER, _L_PER)], idx_l)
        pltpu.sync_copy(sid_hbm.at[pl.ds(w * _S_PER, _S_PER)], idx_s)
        cps = [
            pltpu.async_copy(w_hbm.at[idx_l], rows_l, sem),
            pltpu.async_copy(w_hbm.at[idx_s.at[pl.ds(0, 128)]],
                             rows_s.at[pl.ds(0, 128)], sem),
            pltpu.async_copy(w_hbm.at[idx_s.at[pl.ds(128, 128)]],
                             rows_s.at[pl.ds(128, 128)], sem),
        ]
        cps += [
            pltpu.async_copy(b_hbm.at[idx_l], bias_l, sem),
            pltpu.async_copy(b_hbm.at[idx_s.at[pl.ds(0, 128)]],
                             bias_s.at[pl.ds(0, 128)], sem),
            pltpu.async_copy(b_hbm.at[idx_s.at[pl.ds(128, 128)]],
                             bias_s.at[pl.ds(128, 128)], sem),
        ]
        cps[0].wait()
        o0 = pltpu.async_copy(rows_l, tw_hbm.at[pl.ds(w * _L_PER, _L_PER)], sem2)
        cps[1].wait()
        cps[2].wait()
        o1 = pltpu.async_copy(rows_s, sw_hbm.at[pl.ds(w * _S_PER, _S_PER)], sem2)
        cps[3].wait()
        o2 = pltpu.async_copy(bias_l, tb_hbm.at[pl.ds(w * _L_PER, _L_PER)], sem2)
        cps[4].wait()
        cps[5].wait()
        o3 = pltpu.async_copy(bias_s, sb_hbm.at[pl.ds(w * _S_PER, _S_PER)], sem2)
        for cp in (o0, o1, o2, o3):
            cp.wait()

    return pl.kernel(
        body,
        out_type=(
            jax.ShapeDtypeStruct((_BATCH, _NHID), f32),
            jax.ShapeDtypeStruct((_BATCH,), f32),
            jax.ShapeDtypeStruct((_NSAMPLED, _NHID), f32),
            jax.ShapeDtypeStruct((_NSAMPLED,), f32),
        ),
        mesh=mesh,
        scratch_types=[
            pltpu.VMEM((_L_PER,), i32),
            pltpu.VMEM((_S_PER,), i32),
            pltpu.VMEM((_L_PER, _NHID), f32),
            pltpu.VMEM((_S_PER, _NHID), f32),
            pltpu.VMEM((_L_PER,), f32),
            pltpu.VMEM((_S_PER,), f32),
            pltpu.SemaphoreType.DMA,
            pltpu.SemaphoreType.DMA,
        ],
    )


def _log_expected_count(idx_f):
    p = (jnp.log(idx_f + 2.0) - jnp.log(idx_f + 1.0)) / _LOG_NT1
    return jnp.log(-(jnp.exp(_NSAMPLED * jnp.log(1.0 - p)) - 1.0))


def _mm_body(x_ref, tw_ref, tladj_ref, lab_ref, swp_ref, pk_ref, out_ref):
    x = x_ref[...]
    mm = lax.dot_general(swp_ref[...], x, (((1,), (1,)), ((), ())),
                         preferred_element_type=jnp.float32)
    pk = pk_ref[...]
    sid = lax.bitcast_convert_type(pk[:, 1:2], jnp.int32)
    res = mm + pk[:, :1]
    res = jnp.where(sid == lab_ref[...], jnp.float32(-1e37), res)
    out_ref[...] = res

    @pl.when(pl.program_id(0) == 0)
    def _():
        xtw = x * tw_ref[...]
        tl = lax.dot_general(jnp.ones((1, _NHID), jnp.float32), xtw,
                             (((1,), (1,)), ((), ())),
                             preferred_element_type=jnp.float32)
        out_ref[:1, :] = tl + tladj_ref[...]


def kernel(inputs, labels, W, b, sample_ids):
    f32, i32 = jnp.float32, jnp.int32
    tw, tb, sw, sb = _sc_gather_fn()(W, b, labels, sample_ids)
    swp = jnp.concatenate([jnp.zeros((1, _NHID), f32), sw], axis=0)
    adj = (sb - _log_expected_count(sample_ids.astype(f32))).reshape(-1, 1)
    sid_f = lax.bitcast_convert_type(sample_ids, f32).reshape(-1, 1)
    pk = jnp.concatenate([
        jnp.zeros((1, 2), f32),
        jnp.concatenate([adj, sid_f], axis=1),
    ], axis=0)
    tladj = (tb - _log_expected_count(labels.astype(f32))).reshape(1, _BATCH)
    labc = labels.reshape(1, _BATCH)

    bn = 512
    nsp1 = _NSAMPLED + 1
    out_t = pl.pallas_call(
        _mm_body,
        grid=(pl.cdiv(nsp1, bn),),
        in_specs=[
            pl.BlockSpec((_BATCH, _NHID), lambda i: (0, 0)),
            pl.BlockSpec((_BATCH, _NHID), lambda i: (0, 0)),
            pl.BlockSpec((1, _BATCH), lambda i: (0, 0)),
            pl.BlockSpec((1, _BATCH), lambda i: (0, 0)),
            pl.BlockSpec((bn, _NHID), lambda i: (i, 0)),
            pl.BlockSpec((bn, 2), lambda i: (i, 0)),
        ],
        out_specs=pl.BlockSpec((bn, _BATCH), lambda i: (i, 0)),
        out_shape=jax.ShapeDtypeStruct((nsp1, _BATCH), f32),
    )(inputs, tw, tladj, labc, swp, pk)
    return (out_t.T, jnp.zeros((_BATCH,), i32))


# speedup vs baseline: 1.0124x; 1.0124x over previous
import math

import jax
import jax.numpy as jnp
from jax import lax
from jax.experimental import pallas as pl
from jax.experimental.pallas import tpu as pltpu
from jax.experimental.pallas import tpu_sc as plsc

_NTOKENS = 1000000
_NSAMPLED = 8192
_NHID = 128
_BATCH = 4096
_NC, _NS = 2, 16
_NW = _NC * _NS
_L_PER = _BATCH // _NW
_S_PER = _NSAMPLED // _NW
_LOG_NT1 = math.log(_NTOKENS + 1)


def _sc_gather_fn():
    f32, i32 = jnp.float32, jnp.int32
    mesh = plsc.VectorSubcoreMesh(
        core_axis_name="c", subcore_axis_name="s",
        num_cores=_NC, num_subcores=_NS)

    def body(w_hbm, b_hbm, lab_hbm, sid_hbm,
             tw_hbm, tb_hbm, sw_hbm, sb_hbm,
             idx_l, idx_s, rows_l, rows_s, bias_l, bias_s, sem):
        c = lax.axis_index("c")
        s = lax.axis_index("s")
        w = s * _NC + c
        pltpu.sync_copy(lab_hbm.at[pl.ds(w * _L_PER, _L_PER)], idx_l)
        pltpu.sync_copy(sid_hbm.at[pl.ds(w * _S_PER, _S_PER)], idx_s)
        cps = [
            pltpu.async_copy(w_hbm.at[idx_l], rows_l, sem),
            pltpu.async_copy(w_hbm.at[idx_s.at[pl.ds(0, 128)]],
                             rows_s.at[pl.ds(0, 128)], sem),
            pltpu.async_copy(w_hbm.at[idx_s.at[pl.ds(128, 128)]],
                             rows_s.at[pl.ds(128, 128)], sem),
        ]
        cps += [
            pltpu.async_copy(b_hbm.at[idx_l], bias_l, sem),
            pltpu.async_copy(b_hbm.at[idx_s.at[pl.ds(0, 128)]],
                             bias_s.at[pl.ds(0, 128)], sem),
            pltpu.async_copy(b_hbm.at[idx_s.at[pl.ds(128, 128)]],
                             bias_s.at[pl.ds(128, 128)], sem),
        ]
        for cp in cps:
            cp.wait()
        pltpu.sync_copy(rows_l, tw_hbm.at[pl.ds(w * _L_PER, _L_PER)])
        pltpu.sync_copy(rows_s, sw_hbm.at[pl.ds(w * _S_PER, _S_PER)])
        pltpu.sync_copy(bias_l, tb_hbm.at[pl.ds(w * _L_PER, _L_PER)])
        pltpu.sync_copy(bias_s, sb_hbm.at[pl.ds(w * _S_PER, _S_PER)])

    return pl.kernel(
        body,
        out_type=(
            jax.ShapeDtypeStruct((_BATCH, _NHID), f32),
            jax.ShapeDtypeStruct((_BATCH,), f32),
            jax.ShapeDtypeStruct((_NSAMPLED, _NHID), f32),
            jax.ShapeDtypeStruct((_NSAMPLED,), f32),
        ),
        mesh=mesh,
        scratch_types=[
            pltpu.VMEM((_L_PER,), i32),
            pltpu.VMEM((_S_PER,), i32),
            pltpu.VMEM((_L_PER, _NHID), f32),
            pltpu.VMEM((_S_PER, _NHID), f32),
            pltpu.VMEM((_L_PER,), f32),
            pltpu.VMEM((_S_PER,), f32),
            pltpu.SemaphoreType.DMA,
        ],
    )


def _log_expected_count(idx_f):
    p = (jnp.log(idx_f + 2.0) - jnp.log(idx_f + 1.0)) / _LOG_NT1
    return jnp.log(-(jnp.exp(_NSAMPLED * jnp.log(1.0 - p)) - 1.0))


def _mm_body(x_ref, tw_ref, tladj_ref, lab_ref, swp_ref, pk_ref, out_ref):
    x = x_ref[...]
    mm = lax.dot_general(swp_ref[...], x, (((1,), (1,)), ((), ())),
                         preferred_element_type=jnp.float32)
    pk = pk_ref[...]
    sid = lax.bitcast_convert_type(pk[:, 1:2], jnp.int32)
    res = mm + pk[:, :1]
    res = jnp.where(sid == lab_ref[...], jnp.float32(-1e37), res)
    out_ref[...] = res

    @pl.when(pl.program_id(0) == 0)
    def _():
        xtw = x * tw_ref[...]
        tl = lax.dot_general(jnp.ones((1, _NHID), jnp.float32), xtw,
                             (((1,), (1,)), ((), ())),
                             preferred_element_type=jnp.float32)
        out_ref[:1, :] = tl + tladj_ref[...]


def kernel(inputs, labels, W, b, sample_ids):
    f32, i32 = jnp.float32, jnp.int32
    tw, tb, sw, sb = _sc_gather_fn()(W, b, labels, sample_ids)
    swp = jnp.concatenate([jnp.zeros((1, _NHID), f32), sw], axis=0)
    adj = (sb - _log_expected_count(sample_ids.astype(f32))).reshape(-1, 1)
    sid_f = lax.bitcast_convert_type(sample_ids, f32).reshape(-1, 1)
    pk = jnp.concatenate([
        jnp.zeros((1, 2), f32),
        jnp.concatenate([adj, sid_f], axis=1),
    ], axis=0)
    tladj = (tb - _log_expected_count(labels.astype(f32))).reshape(1, _BATCH)
    labc = labels.reshape(1, _BATCH)

    bn = 512
    nsp1 = _NSAMPLED + 1
    out_t = pl.pallas_call(
        _mm_body,
        grid=(pl.cdiv(nsp1, bn),),
        in_specs=[
            pl.BlockSpec((_BATCH, _NHID), lambda i: (0, 0)),
            pl.BlockSpec((_BATCH, _NHID), lambda i: (0, 0)),
            pl.BlockSpec((1, _BATCH), lambda i: (0, 0)),
            pl.BlockSpec((1, _BATCH), lambda i: (0, 0)),
            pl.BlockSpec((bn, _NHID), lambda i: (i, 0)),
            pl.BlockSpec((bn, 2), lambda i: (i, 0)),
        ],
        out_specs=pl.BlockSpec((bn, _BATCH), lambda i: (i, 0)),
        out_shape=jax.ShapeDtypeStruct((nsp1, _BATCH), f32),
    )(inputs, tw, tladj, labc, swp, pk)
    return (out_t.T, jnp.zeros((_BATCH,), i32))
